# baseline (speedup 1.0000x reference)
import functools

import jax
import jax.numpy as jnp
from jax import lax
from jax.experimental import pallas as pl
from jax.experimental.pallas import tpu as pltpu
from jax.experimental.pallas import tpu_sc as plsc

_TOP_K = 8
_LANES = 16


def _gate_body(x_ref, w_ref, topv_ref, topi_ref, loss_ref, acc_ref,
               *, tokens_total, num_experts):
    i = pl.program_id(0)
    logits = jax.lax.dot_general(
        x_ref[...], w_ref[...],
        dimension_numbers=(((1,), (1,)), ((), ())),
        preferred_element_type=jnp.float32)
    m = jnp.max(logits, axis=1, keepdims=True)
    e = jnp.exp(logits - m)
    s = jnp.sum(e, axis=1, keepdims=True)
    w = e / s

    colsum = jnp.sum(w, axis=0, keepdims=True)

    @pl.when(i == 0)
    def _():
        acc_ref[...] = colsum

    @pl.when(i > 0)
    def _():
        acc_ref[...] = acc_ref[...] + colsum

    wk = w.T
    row = jax.lax.broadcasted_iota(jnp.int32, wk.shape, 0)
    for j in range(_TOP_K):
        mx = jnp.max(wk, axis=0, keepdims=True)
        ismax = wk == mx
        idxv = jnp.min(jnp.where(ismax, row, num_experts), axis=0,
                       keepdims=True)
        sel = row == idxv
        wk = jnp.where(sel, -1.0, wk)
        topv_ref[pl.ds(j, 1), :] = mx
        topi_ref[pl.ds(j, 1), :] = idxv

    @pl.when(i == pl.num_programs(0) - 1)
    def _():
        frac = acc_ref[...] / tokens_total
        target = 1.0 / num_experts
        loss_ref[0, 0] = jnp.sum((frac - target) ** 2) * num_experts


def _make_scatter_kernel(tokens, num_experts):
    info = plsc.get_sparse_core_info()
    nw = info.num_cores * info.num_subcores
    t_per_w = tokens // nw
    groups = t_per_w // _LANES
    mesh = plsc.VectorSubcoreMesh(core_axis_name="c", subcore_axis_name="s")

    @functools.partial(
        pl.kernel, mesh=mesh,
        compiler_params=pltpu.CompilerParams(needs_layout_passes=False),
        out_type=[
            jax.ShapeDtypeStruct((tokens * num_experts,), jnp.float32),
            jax.ShapeDtypeStruct((tokens * _TOP_K,), jnp.int32),
        ],
        scratch_types=[
            pltpu.VMEM((_TOP_K, t_per_w), jnp.float32),
            pltpu.VMEM((_TOP_K, t_per_w), jnp.int32),
            pltpu.VMEM((t_per_w * num_experts,), jnp.float32),
            pltpu.VMEM((t_per_w * _TOP_K,), jnp.int32),
            pltpu.SemaphoreType.DMA,
            pltpu.SemaphoreType.DMA,
        ],
    )
    def sc_scatter(topv_hbm, topi_hbm, sparse_hbm, tidx_hbm,
                   tv_v, ti_v, sp_v, to_v, in_sem, out_sem):
        wid = lax.axis_index("s") * info.num_cores + lax.axis_index("c")
        base = wid * t_per_w
        in0 = pltpu.async_copy(topv_hbm.at[:, pl.ds(base, t_per_w)], tv_v,
                               in_sem)
        in1 = pltpu.async_copy(topi_hbm.at[:, pl.ds(base, t_per_w)], ti_v,
                               in_sem)

        zeros = jnp.zeros((_LANES,), jnp.float32)

        def _zero_body(i, carry):
            for k in range(64):
                sp_v[pl.ds(i * 1024 + k * _LANES, _LANES)] = zeros
            return carry

        lax.fori_loop(0, t_per_w * num_experts // 1024, _zero_body, 0)
        in0.wait()
        in1.wait()

        lane = jax.lax.iota(jnp.int32, _LANES)
        out_copies = []

        for g in range(groups):
            t0 = g * _LANES
            vs = [tv_v[j, pl.ds(t0, _LANES)] for j in range(_TOP_K)]
            s01, s23 = vs[0] + vs[1], vs[2] + vs[3]
            s45, s67 = vs[4] + vs[5], vs[6] + vs[7]
            tot = (s01 + s23) + (s45 + s67)
            inv = 1.0 / (tot + 1e-9)
            row = (t0 + lane) * num_experts
            rowk = (t0 + lane) * _TOP_K
            for j in range(_TOP_K):
                idx = ti_v[j, pl.ds(t0, _LANES)]
                plsc.store_scatter(sp_v, [row + idx], vs[j] * inv)
                plsc.store_scatter(to_v, [rowk + j], idx)
            if g % 4 == 3:
                t0b = (g - 3) * _LANES
                sp_lo = t0b * num_experts
                to_lo = t0b * _TOP_K
                out_copies.append(pltpu.async_copy(
                    sp_v.at[pl.ds(sp_lo, 4 * _LANES * num_experts)],
                    sparse_hbm.at[pl.ds(base * num_experts + sp_lo,
                                        4 * _LANES * num_experts)],
                    out_sem))
                out_copies.append(pltpu.async_copy(
                    to_v.at[pl.ds(to_lo, 4 * _LANES * _TOP_K)],
                    tidx_hbm.at[pl.ds(base * _TOP_K + to_lo,
                                      4 * _LANES * _TOP_K)],
                    out_sem))

        for c in out_copies:
            c.wait()

    return sc_scatter


def kernel(x, W, training):
    del training
    tokens, d_model = x.shape
    num_experts = W.shape[0]
    block = 1024
    grid = tokens // block

    topv, topi, loss = pl.pallas_call(
        functools.partial(_gate_body, tokens_total=tokens,
                          num_experts=num_experts),
        grid=(grid,),
        in_specs=[
            pl.BlockSpec((block, d_model), lambda i: (i, 0)),
            pl.BlockSpec((num_experts, d_model), lambda i: (0, 0)),
        ],
        out_specs=[
            pl.BlockSpec((_TOP_K, block), lambda i: (0, i)),
            pl.BlockSpec((_TOP_K, block), lambda i: (0, i)),
            pl.BlockSpec(memory_space=pltpu.SMEM),
        ],
        out_shape=[
            jax.ShapeDtypeStruct((_TOP_K, tokens), jnp.float32),
            jax.ShapeDtypeStruct((_TOP_K, tokens), jnp.int32),
            jax.ShapeDtypeStruct((1, 1), jnp.float32),
        ],
        scratch_shapes=[pltpu.VMEM((1, num_experts), jnp.float32)],
        compiler_params=pltpu.CompilerParams(
            vmem_limit_bytes=100 * 1024 * 1024),
    )(x, W)

    sparse_flat, tidx_flat = _make_scatter_kernel(tokens, num_experts)(
        topv, topi)
    sparse = sparse_flat.reshape(tokens, num_experts)
    tidx = tidx_flat.reshape(tokens, _TOP_K)
    return (sparse, tidx, loss[0, 0])
